# baseline (speedup 1.0000x reference)
import functools

import jax
import jax.numpy as jnp
from jax import lax
from jax.experimental import pallas as pl
from jax.experimental.pallas import tpu as pltpu
from jax.experimental.pallas import tpu_sc as plsc

_W = 256


@functools.cache
def _make_gather(V, D, B, H):
    info = plsc.get_sparse_core_info()
    NC, NS, L = info.num_cores, info.num_subcores, info.num_lanes
    NW = NC * NS
    N = B * H
    assert D == 64 and L == 16
    assert B % _W == 0
    chunks_per_h = B // _W
    n_chunks = N // _W
    assert n_chunks % NW == 0
    nc_per_w = n_chunks // NW
    mesh = plsc.VectorSubcoreMesh(core_axis_name="c", subcore_axis_name="s")

    @functools.partial(
        pl.kernel,
        mesh=mesh,
        out_type=jax.ShapeDtypeStruct((H, D, B), jnp.float32),
        scratch_types=[
            pltpu.VMEM((_W,), jnp.int32),
            pltpu.VMEM((_W,), jnp.int32),
            pltpu.VMEM((_W,), jnp.int32),
            pltpu.VMEM((_W,), jnp.int32),
            pltpu.VMEM((2, _W, 2 * D), jnp.float32),
            pltpu.VMEM((2, D, _W), jnp.float32),
            pltpu.SemaphoreType.DMA,
            pltpu.SemaphoreType.DMA,
            pltpu.SemaphoreType.DMA,
            pltpu.SemaphoreType.DMA,
        ],
        compiler_params=pltpu.CompilerParams(
            use_tc_tiling_on_sc=True, needs_layout_passes=False),
    )
    def gather_kernel(idx_hbm, t2_hbm, out_hbm, idx_v0, idx_v1,
                      idx2_v0, idx2_v1, g_v, o_v,
                      gsem0, gsem1, wsem0, wsem1):
        idx_v = (idx_v0, idx_v1)
        idx2_v = (idx2_v0, idx2_v1)
        gsem = (gsem0, gsem1)
        wsem = (wsem0, wsem1)
        wid = lax.axis_index("s") * NC + lax.axis_index("c")
        c0 = wid * nc_per_w

        def stage_idx(i, b):
            off = pl.multiple_of((c0 + i) * _W, _W)
            pltpu.sync_copy(idx_hbm.at[pl.ds(off, _W)], idx_v[b])
            for jj in range(_W // L):
                v = idx_v[b][pl.ds(jj * L, L)]
                idx2_v[b][pl.ds(jj * L, L)] = lax.shift_right_logical(v, 1)

        def start_gather(b):
            pltpu.async_copy(t2_hbm.at[idx2_v[b]], g_v.at[b], gsem[b])

        def wait_gather(b):
            pltpu.make_async_copy(
                t2_hbm.at[idx2_v[b]], g_v.at[b], gsem[b]).wait()

        def shuffle(b):
            g2 = g_v.at[b]
            def body(jj, carry):
                joff = pl.multiple_of(jj * L, L)
                jv = lax.iota(jnp.int32, L) + jj * L
                par = lax.bitwise_and(idx_v[b][pl.ds(joff, L)], 1)
                pcv = par * D
                for d in range(D):
                    r = plsc.load_gather(g2, [jv, pcv + d])
                    o_v[b, d, pl.ds(joff, L)] = r
                return carry
            lax.fori_loop(0, _W // L, body, 0)

        def start_write(i, b):
            c = c0 + i
            h = c // chunks_per_h
            b0 = pl.multiple_of((c % chunks_per_h) * _W, _W)
            pltpu.async_copy(
                o_v.at[b], out_hbm.at[h, slice(None), pl.ds(b0, _W)], wsem[b])

        def wait_write(b):
            pltpu.make_async_copy(
                o_v.at[b], out_hbm.at[0, slice(None), pl.ds(0, _W)],
                wsem[b]).wait()

        stage_idx(0, 0)
        start_gather(0)

        def body(g, carry):
            for b in range(2):
                i = g * 2 + b

                @pl.when(i + 1 < nc_per_w)
                def _():
                    stage_idx(i + 1, 1 - b)
                    start_gather(1 - b)

                wait_gather(b)

                @pl.when(i >= 2)
                def _():
                    wait_write(b)

                shuffle(b)
                start_write(i, b)
            return carry

        lax.fori_loop(0, nc_per_w // 2, body, 0)
        wait_write(0)
        wait_write(1)

    return gather_kernel


def kernel(x, table):
    B, H = x.shape
    V, D = table.shape
    idx = x.T.reshape(B * H).astype(jnp.int32)
    t2 = table.reshape(V // 2, 2 * D)
    out_phys = _make_gather(V, D, B, H)(idx, t2)
    return jnp.transpose(out_phys, (2, 0, 1))
